# baseline (speedup 1.0000x reference)
import functools

import jax
import jax.numpy as jnp
from jax import lax
from jax.experimental import pallas as pl
from jax.experimental.pallas import tpu as pltpu
from jax.experimental.pallas import tpu_sc as plsc

B = 16384
D = 64
DP = 128
OUTW = 3 + 3 * D
NC = 2
NS = 16
L = 16
NW = NC * NS
BPW = B // NW
CH = 128
NCHUNK = BPW // CH
NG = CH // L

MAX_NORM = 0.95
HALF_LN2 = 0.34657359027997264

_GATHER_DNUMS = lax.GatherDimensionNumbers(
    offset_dims=(), collapsed_slice_dims=(0,), start_index_map=(0,))


def _permute(v, idx):
    return lax.gather(v, idx[:, None], dimension_numbers=_GATHER_DNUMS,
                      slice_sizes=(1,),
                      mode=lax.GatherScatterMode.PROMISE_IN_BOUNDS)


def _combine(a, b, k):
    pa = _permute(a, lax.iota(jnp.int32, L) ^ k)
    pb = _permute(b, lax.iota(jnp.int32, L) ^ k)
    bit0 = (lax.iota(jnp.int32, L) & k) == 0
    return jnp.where(bit0, a, pb) + jnp.where(bit0, pa, b)


def _bitrev_iota():
    l = lax.iota(jnp.int32, L)
    return ((l & 1) << 3) | ((l & 2) << 1) | ((l & 4) >> 1) | ((l & 8) >> 3)


def _hsum16(ps):
    v = [_combine(ps[2 * i], ps[2 * i + 1], 8) for i in range(8)]
    v = [_combine(v[2 * i], v[2 * i + 1], 4) for i in range(4)]
    v = [_combine(v[2 * i], v[2 * i + 1], 2) for i in range(2)]
    s = _combine(v[0], v[1], 1)
    return _permute(s, _bitrev_iota())


def _hsum1(v):
    for k in (1, 2, 4, 8):
        v = v + _permute(v, lax.iota(jnp.int32, L) ^ k)
    return v


def _rsqrt(x):
    i = plsc.bitcast(x, jnp.int32)
    i = 0x5F3759DF - (i >> 1)
    y = plsc.bitcast(i, jnp.float32)
    for _ in range(3):
        y = y * (1.5 - 0.5 * x * y * y)
    return y


def _atanh(x):
    r = (1.0 + x) / (1.0 - x)
    i = plsc.bitcast(r, jnp.int32)
    e = (i >> 23) - 127
    m = plsc.bitcast((i & 0x007FFFFF) | 0x3F800000, jnp.float32)
    z = (m - 1.0) / (m + 1.0)
    z2 = z * z
    p = z * (1.0 + z2 * (1.0 / 3.0 + z2 * (1.0 / 5.0 + z2 * (
        1.0 / 7.0 + z2 * (1.0 / 9.0 + z2 * (1.0 / 11.0))))))
    return HALF_LN2 * e.astype(jnp.float32) + p


def _row_factor(ssq):
    nrm = ssq * _rsqrt(jnp.maximum(ssq, 1e-30))
    nrm = jnp.maximum(nrm, 1e-9)
    scale = jnp.minimum(MAX_NORM / nrm, 1.0)
    xn = jnp.maximum(nrm * scale, 1e-9)
    return scale * (_atanh(xn) / xn)


def _body(numeric, wf, wp, wa, fidx, pidx, aidx, out,
          fi_v, pi_v, ai_v, num_v, anc_v,
          fb0, fb1, pb0, pb1, out_v,
          semf0, semf1, semp0, semp1):
    wid = lax.axis_index("s") * NC + lax.axis_index("c")
    base = wid * BPW

    pltpu.sync_copy(fidx.at[pl.ds(base, BPW)], fi_v)
    pltpu.sync_copy(pidx.at[pl.ds(base, BPW)], pi_v)
    pltpu.sync_copy(aidx.at[pl.ds(base, BPW)], ai_v)
    pltpu.sync_copy(numeric.at[pl.ds(base * 3, BPW * 3)], num_v)
    pltpu.sync_copy(wa, anc_v)

    fbufs, pbufs = (fb0, fb1), (pb0, pb1)
    fsems, psems = (semf0, semf1), (semp0, semp1)

    def fire(c):
        sl = pl.ds(c * CH, CH)
        s = c % 2
        return (pltpu.async_copy(wf.at[fi_v.at[sl]], fbufs[s], fsems[s]),
                pltpu.async_copy(wp.at[pi_v.at[sl]], pbufs[s], psems[s]))

    with jax.named_scope("sc_fire01"):
        cps = {0: fire(0), 1: fire(1)}

    with jax.named_scope("sc_anchor_prep"):
        a0 = [anc_v[0, pl.ds(j * L, L)] for j in range(D // L)]
        a1 = [anc_v[1, pl.ds(j * L, L)] for j in range(D // L)]
        ssq0 = _hsum1(sum(v * v for v in a0))
        ssq1 = _hsum1(sum(v * v for v in a1))
        f0, f1 = _row_factor(ssq0), _row_factor(ssq1)
        anc_base = [v * f1 for v in a1]
        anc_diff = [v0 * f0 - v1 * f1 for v0, v1 in zip(a0, a1)]

    for c in range(NCHUNK):
        s = c % 2
        fbuf, pbuf = fbufs[s], pbufs[s]
        with jax.named_scope("sc_drain"):
            for cp in cps.pop(c):
                cp.wait()

        def group_body(g, gcarry, c=c, fbuf=fbuf, pbuf=pbuf):
            g16 = g * L
            lrows = g16 + lax.iota(jnp.int32, L)
            grows = c * CH + lrows
            orow = lrows * OUTW
            with jax.named_scope("sc_numeric"):
                for ccol in range(3):
                    v = plsc.load_gather(num_v, [grows * 3 + ccol])
                    plsc.store_scatter(out_v, [orow + ccol], v)
            for buf, off in ((fbuf, 3), (pbuf, 3 + D)):
                with jax.named_scope("sc_pass1"):
                    parts = []
                    for r in range(L):
                        p = None
                        for j in range(D // L):
                            v = buf[g16 + r, pl.ds(j * L, L)]
                            sq = v * v
                            p = sq if p is None else p + sq
                        parts.append(p)
                with jax.named_scope("sc_factor"):
                    fac = _row_factor(_hsum16(parts))
                with jax.named_scope("sc_pass2"):
                    for r in range(L):
                        m = _permute(fac, jnp.full((L,), r, jnp.int32))
                        robase = (g16 + r) * OUTW + off
                        for j in range(D // L):
                            v = buf[g16 + r, pl.ds(j * L, L)]
                            out_v[pl.ds(robase + j * L, L)] = v * m
            with jax.named_scope("sc_anchor"):
                av = plsc.load_gather(ai_v, [grows])
                is0 = jnp.where(av == 0, 1.0, 0.0).astype(jnp.float32)
                for r in range(L):
                    s0 = _permute(is0, jnp.full((L,), r, jnp.int32))
                    robase = (g16 + r) * OUTW + 3 + 2 * D
                    for j in range(D // L):
                        out_v[pl.ds(robase + j * L, L)] = (
                            anc_base[j] + anc_diff[j] * s0)
            return gcarry

        lax.fori_loop(0, NG, group_body, 0)
        if c + 2 < NCHUNK:
            with jax.named_scope("sc_fire_next"):
                cps[c + 2] = fire(c + 2)
        with jax.named_scope("sc_flush"):
            row0 = pl.multiple_of((base + c * CH) * OUTW, 8)
            pltpu.sync_copy(out_v, out.at[pl.ds(row0, CH * OUTW)])


@jax.jit
def kernel(numeric, W_family, W_projection, W_anchor,
           family_idx, projection_idx, anchor_idx):
    mesh = plsc.VectorSubcoreMesh(
        core_axis_name="c", subcore_axis_name="s",
        num_cores=NC, num_subcores=NS)
    enc = pl.kernel(
        _body,
        out_type=jax.ShapeDtypeStruct((B * OUTW,), jnp.float32),
        mesh=mesh,
        scratch_types=[
            pltpu.VMEM((BPW,), jnp.int32),
            pltpu.VMEM((BPW,), jnp.int32),
            pltpu.VMEM((BPW,), jnp.int32),
            pltpu.VMEM((BPW * 3,), jnp.float32),
            pltpu.VMEM((2, D), jnp.float32),
            pltpu.VMEM((CH, D), jnp.float32),
            pltpu.VMEM((CH, D), jnp.float32),
            pltpu.VMEM((CH, D), jnp.float32),
            pltpu.VMEM((CH, D), jnp.float32),
            pltpu.VMEM((CH * OUTW,), jnp.float32),
            pltpu.SemaphoreType.DMA,
            pltpu.SemaphoreType.DMA,
            pltpu.SemaphoreType.DMA,
            pltpu.SemaphoreType.DMA,
        ],
        compiler_params=pltpu.CompilerParams(
            needs_layout_passes=False, use_tc_tiling_on_sc=False),
    )
    flat = enc(numeric.reshape(B * 3), W_family, W_projection, W_anchor,
               family_idx, projection_idx, anchor_idx)
    return flat.reshape(B, OUTW)


# speedup vs baseline: 1.1193x; 1.1193x over previous
import functools

import jax
import jax.numpy as jnp
from jax import lax
from jax.experimental import pallas as pl
from jax.experimental.pallas import tpu as pltpu
from jax.experimental.pallas import tpu_sc as plsc

B = 16384
D = 64
DP = 128
OUTW = 3 + 3 * D
NC = 2
NS = 16
L = 16
NW = NC * NS
BPW = B // NW
CH = 128
NCHUNK = BPW // CH
NG = CH // L

MAX_NORM = 0.95
HALF_LN2 = 0.34657359027997264

_GATHER_DNUMS = lax.GatherDimensionNumbers(
    offset_dims=(), collapsed_slice_dims=(0,), start_index_map=(0,))


def _permute(v, idx):
    return lax.gather(v, idx[:, None], dimension_numbers=_GATHER_DNUMS,
                      slice_sizes=(1,),
                      mode=lax.GatherScatterMode.PROMISE_IN_BOUNDS)


def _combine(a, b, k):
    pa = _permute(a, lax.iota(jnp.int32, L) ^ k)
    pb = _permute(b, lax.iota(jnp.int32, L) ^ k)
    bit0 = (lax.iota(jnp.int32, L) & k) == 0
    return jnp.where(bit0, a, pb) + jnp.where(bit0, pa, b)


def _bitrev_iota():
    l = lax.iota(jnp.int32, L)
    return ((l & 1) << 3) | ((l & 2) << 1) | ((l & 4) >> 1) | ((l & 8) >> 3)


def _hsum16(ps):
    v = [_combine(ps[2 * i], ps[2 * i + 1], 8) for i in range(8)]
    v = [_combine(v[2 * i], v[2 * i + 1], 4) for i in range(4)]
    v = [_combine(v[2 * i], v[2 * i + 1], 2) for i in range(2)]
    s = _combine(v[0], v[1], 1)
    return _permute(s, _bitrev_iota())


def _hsum1(v):
    for k in (1, 2, 4, 8):
        v = v + _permute(v, lax.iota(jnp.int32, L) ^ k)
    return v


def _rsqrt(x):
    i = plsc.bitcast(x, jnp.int32)
    i = 0x5F3759DF - (i >> 1)
    y = plsc.bitcast(i, jnp.float32)
    for _ in range(3):
        y = y * (1.5 - 0.5 * x * y * y)
    return y


def _atanh(x):
    r = (1.0 + x) / (1.0 - x)
    i = plsc.bitcast(r, jnp.int32)
    e = (i >> 23) - 127
    m = plsc.bitcast((i & 0x007FFFFF) | 0x3F800000, jnp.float32)
    z = (m - 1.0) / (m + 1.0)
    z2 = z * z
    p = z * (1.0 + z2 * (1.0 / 3.0 + z2 * (1.0 / 5.0 + z2 * (
        1.0 / 7.0 + z2 * (1.0 / 9.0 + z2 * (1.0 / 11.0))))))
    return HALF_LN2 * e.astype(jnp.float32) + p


def _row_factor(ssq):
    nrm = ssq * _rsqrt(jnp.maximum(ssq, 1e-30))
    nrm = jnp.maximum(nrm, 1e-9)
    scale = jnp.minimum(MAX_NORM / nrm, 1.0)
    xn = jnp.maximum(nrm * scale, 1e-9)
    return scale * (_atanh(xn) / xn)


def _body(numeric, wf, wp, wa, fidx, pidx, aidx, out,
          fi_v, pi_v, ai_v, num_v, anc_v,
          fb0, fb1, pb0, pb1, out_v,
          semf0, semf1, semp0, semp1):
    wid = lax.axis_index("s") * NC + lax.axis_index("c")
    base = wid * BPW

    pltpu.sync_copy(fidx.at[pl.ds(base, BPW)], fi_v)
    pltpu.sync_copy(pidx.at[pl.ds(base, BPW)], pi_v)
    pltpu.sync_copy(aidx.at[pl.ds(base, BPW)], ai_v)
    pltpu.sync_copy(numeric.at[pl.ds(base * 3, BPW * 3)], num_v)
    pltpu.sync_copy(wa, anc_v)

    fbufs, pbufs = (fb0, fb1), (pb0, pb1)
    fsems, psems = (semf0, semf1), (semp0, semp1)

    def fire(c):
        sl = pl.ds(c * CH, CH)
        s = c % 2
        return (pltpu.async_copy(wf.at[fi_v.at[sl]], fbufs[s], fsems[s]),
                pltpu.async_copy(wp.at[pi_v.at[sl]], pbufs[s], psems[s]))

    with jax.named_scope("sc_fire01"):
        cps = {0: fire(0), 1: fire(1)}

    with jax.named_scope("sc_anchor_prep"):
        a0 = [anc_v[0, pl.ds(j * L, L)] for j in range(D // L)]
        a1 = [anc_v[1, pl.ds(j * L, L)] for j in range(D // L)]
        ssq0 = _hsum1(sum(v * v for v in a0))
        ssq1 = _hsum1(sum(v * v for v in a1))
        f0, f1 = _row_factor(ssq0), _row_factor(ssq1)
        anc_base = [v * f1 for v in a1]
        anc_diff = [v0 * f0 - v1 * f1 for v0, v1 in zip(a0, a1)]

    for c in range(NCHUNK):
        s = c % 2
        fbuf, pbuf = fbufs[s], pbufs[s]
        with jax.named_scope("sc_drain"):
            for cp in cps.pop(c):
                cp.wait()

        def group_body(g, gcarry, c=c, fbuf=fbuf, pbuf=pbuf):
            g16 = g * L
            lrows = g16 + lax.iota(jnp.int32, L)
            grows = c * CH + lrows
            orow = lrows * OUTW
            with jax.named_scope("sc_numeric"):
                for ccol in range(3):
                    v = plsc.load_gather(num_v, [grows * 3 + ccol])
                    plsc.store_scatter(out_v, [orow + ccol], v)
            with jax.named_scope("sc_pass1"):
                parts_f, parts_p = [], []
                for r in range(L):
                    pf = pp = None
                    for j in range(D // L):
                        vf = fbuf[g16 + r, pl.ds(j * L, L)]
                        vp = pbuf[g16 + r, pl.ds(j * L, L)]
                        sf, sp = vf * vf, vp * vp
                        pf = sf if pf is None else pf + sf
                        pp = sp if pp is None else pp + sp
                    parts_f.append(pf)
                    parts_p.append(pp)
            with jax.named_scope("sc_factor"):
                fac_f = _row_factor(_hsum16(parts_f))
                fac_p = _row_factor(_hsum16(parts_p))
                av = plsc.load_gather(ai_v, [grows])
                is0 = jnp.where(av == 0, 1.0, 0.0).astype(jnp.float32)
            with jax.named_scope("sc_pass2"):
                for r in range(L):
                    rv = jnp.full((L,), r, jnp.int32)
                    mf = _permute(fac_f, rv)
                    mp = _permute(fac_p, rv)
                    s0 = _permute(is0, rv)
                    robase = (g16 + r) * OUTW + 3
                    for j in range(D // L):
                        vf = fbuf[g16 + r, pl.ds(j * L, L)]
                        vp = pbuf[g16 + r, pl.ds(j * L, L)]
                        out_v[pl.ds(robase + j * L, L)] = vf * mf
                        out_v[pl.ds(robase + D + j * L, L)] = vp * mp
                        out_v[pl.ds(robase + 2 * D + j * L, L)] = (
                            anc_base[j] + anc_diff[j] * s0)
            return gcarry

        lax.fori_loop(0, NG, group_body, 0)
        if c + 2 < NCHUNK:
            with jax.named_scope("sc_fire_next"):
                cps[c + 2] = fire(c + 2)
        with jax.named_scope("sc_flush"):
            row0 = pl.multiple_of((base + c * CH) * OUTW, 8)
            pltpu.sync_copy(out_v, out.at[pl.ds(row0, CH * OUTW)])


@jax.jit
def kernel(numeric, W_family, W_projection, W_anchor,
           family_idx, projection_idx, anchor_idx):
    mesh = plsc.VectorSubcoreMesh(
        core_axis_name="c", subcore_axis_name="s",
        num_cores=NC, num_subcores=NS)
    enc = pl.kernel(
        _body,
        out_type=jax.ShapeDtypeStruct((B * OUTW,), jnp.float32),
        mesh=mesh,
        scratch_types=[
            pltpu.VMEM((BPW,), jnp.int32),
            pltpu.VMEM((BPW,), jnp.int32),
            pltpu.VMEM((BPW,), jnp.int32),
            pltpu.VMEM((BPW * 3,), jnp.float32),
            pltpu.VMEM((2, DP), jnp.float32),
            pltpu.VMEM((CH, DP), jnp.float32),
            pltpu.VMEM((CH, DP), jnp.float32),
            pltpu.VMEM((CH, DP), jnp.float32),
            pltpu.VMEM((CH, DP), jnp.float32),
            pltpu.VMEM((CH * OUTW,), jnp.float32),
            pltpu.SemaphoreType.DMA,
            pltpu.SemaphoreType.DMA,
            pltpu.SemaphoreType.DMA,
            pltpu.SemaphoreType.DMA,
        ],
        compiler_params=pltpu.CompilerParams(
            needs_layout_passes=False, use_tc_tiling_on_sc=True),
    )
    pad = ((0, 0), (0, DP - D))
    flat = enc(numeric.reshape(B * 3),
               jnp.pad(W_family, pad), jnp.pad(W_projection, pad),
               jnp.pad(W_anchor, pad),
               family_idx, projection_idx, anchor_idx)
    return flat.reshape(B, OUTW)
